# baseline (speedup 1.0000x reference)
import functools

import jax
import jax.numpy as jnp
import numpy as np
from jax import lax
from jax.experimental import pallas as pl
from jax.experimental.pallas import tpu as pltpu
from jax.experimental.pallas import tpu_sc as plsc

B, C, H, W = 4, 96, 224, 224
OH, OW = H // 2, W // 2
L = OH * OW
BC = B * C

NSC = 96
NTC = BC - NSC
ROWS = 8
GRID = NTC // ROWS

_KS1 = 42
_KS2 = 0x1BD11BDA ^ 42
_TINY = np.float32(np.finfo(np.float32).tiny)


def _rotl(x, r):
    return (x << r) | lax.shift_right_logical(x, 32 - r)


def _rounds(x0, x1, rots):
    for r in rots:
        x0 = x0 + x1
        x1 = x0 ^ _rotl(x1, r)
    return x0, x1


def _threefry2x32(x0, x1):
    rot_a = (13, 15, 26, 6)
    rot_b = (17, 29, 16, 24)
    x1 = x1 + _KS1
    x0, x1 = _rounds(x0, x1, rot_a)
    x0, x1 = x0 + _KS1, x1 + (_KS2 + 1)
    x0, x1 = _rounds(x0, x1, rot_b)
    x0, x1 = x0 + _KS2, x1 + 2
    x0, x1 = _rounds(x0, x1, rot_a)
    x0, x1 = x0, x1 + (_KS1 + 3)
    x0, x1 = _rounds(x0, x1, rot_b)
    x0, x1 = x0 + _KS1, x1 + (_KS2 + 4)
    x0, x1 = _rounds(x0, x1, rot_a)
    x0, x1 = x0 + _KS2, x1 + 5
    return x0, x1


def _gumbel(cnt):
    o0, o1 = _threefry2x32(jnp.int32(0), cnt)
    bits = o0 ^ o1
    mant = lax.shift_right_logical(bits, 9) | 0x3F800000
    f = lax.bitcast_convert_type(mant, jnp.float32)
    u = jnp.maximum(f - 1.0, _TINY)
    return -jnp.log(-jnp.log(u))


_OHC = OH // 2


def _pool_kernel(x_ref, out_ref):
    i = pl.program_id(0)
    sh = (ROWS, _OHC, W)
    bc = lax.broadcasted_iota(jnp.int32, sh, 0) + i * ROWS
    ohi = lax.broadcasted_iota(jnp.int32, sh, 1)
    cc = lax.broadcasted_iota(jnp.int32, sh, 2)
    for h in range(2):
        oh = ohi + h * _OHC
        base = ((bc * OH + oh) * OW + lax.shift_right_logical(cc, 1)) * 4 + (cc & 1)
        sl = slice(h * _OHC, (h + 1) * _OHC)
        v0 = x_ref[:, sl, 0, :]
        v1 = x_ref[:, sl, 1, :]
        s0 = v0 + _gumbel(base)
        s1 = v1 + _gumbel(base + 2)
        cr = s1 > s0
        sr = jnp.where(cr, s1, s0)
        vr = jnp.where(cr, v1, v0)
        ss = pltpu.roll(sr, W - 1, 2)
        vs = pltpu.roll(vr, W - 1, 2)
        c = ss > sr
        win = jnp.where(c, vs, vr)
        for b in range(7):
            mask = (cc & (1 << b)) != 0
            win = jnp.where(mask, pltpu.roll(win, W - (1 << b), 2), win)
        out_ref[:, sl, :] = win[:, :, :OW]


_NC, _NS, _LANES = 2, 16, 16
_NW = _NC * _NS
_PER_W = NSC // _NW

_LN2 = np.float32(0.6931471805599453)
_SQRT2 = np.float32(1.4142135623730951)


def _sc_log(x):
    bi = lax.bitcast_convert_type(x, jnp.int32)
    e = lax.shift_right_logical(bi, 23) - 127
    m = lax.bitcast_convert_type((bi & 0x7FFFFF) | 0x3F800000, jnp.float32)
    big = m > _SQRT2
    m = jnp.where(big, m * np.float32(0.5), m)
    e = jnp.where(big, e + 1, e)
    z = (m - np.float32(1.0)) / (m + np.float32(1.0))
    z2 = z * z
    p = np.float32(2.0 / 11.0)
    p = p * z2 + np.float32(2.0 / 9.0)
    p = p * z2 + np.float32(2.0 / 7.0)
    p = p * z2 + np.float32(2.0 / 5.0)
    p = p * z2 + np.float32(2.0 / 3.0)
    p = p * z2 + np.float32(2.0)
    return e.astype(jnp.float32) * _LN2 + p * z


def _sc_gumbel(cnt):
    o0, o1 = _threefry2x32(jnp.int32(0), cnt)
    bits = o0 ^ o1
    mant = lax.shift_right_logical(bits, 9) | 0x3F800000
    f = lax.bitcast_convert_type(mant, jnp.float32)
    u = jnp.maximum(f - np.float32(1.0), _TINY)
    return -_sc_log(-_sc_log(u))


def _sc_pool_kernel(x_hbm, out_hbm, img_v, out_v, sem):
    wid = lax.axis_index("s") * _NC + lax.axis_index("c")
    lane = lax.iota(jnp.int32, _LANES)
    for r in range(_PER_W):
        bc = NTC + wid * _PER_W + r
        pltpu.sync_copy(x_hbm.at[bc], img_v)

        def body(oh, _):
            for k in range(OW // _LANES):
                ow = lane + k * _LANES
                f0 = (2 * oh) * W + ow * 2
                v0 = plsc.load_gather(img_v, [f0])
                v1 = plsc.load_gather(img_v, [f0 + 1])
                v2 = plsc.load_gather(img_v, [f0 + W])
                v3 = plsc.load_gather(img_v, [f0 + (W + 1)])
                base = ((bc * OH + oh) * OW + ow) * 4
                bs = v0 + _sc_gumbel(base)
                bv = v0
                for j, vj in ((1, v1), (2, v2), (3, v3)):
                    sj = vj + _sc_gumbel(base + j)
                    cj = sj > bs
                    bs = jnp.where(cj, sj, bs)
                    bv = jnp.where(cj, vj, bv)
                out_v[pl.ds(oh * OW + k * _LANES, _LANES)] = bv
            return _

        lax.fori_loop(0, OH, body, None)
        pltpu.sync_copy(out_v, out_hbm.at[wid * _PER_W + r])


@functools.cache
def _sc_pool():
    return functools.partial(
        pl.kernel,
        mesh=plsc.VectorSubcoreMesh(core_axis_name="c", subcore_axis_name="s"),
        compiler_params=pltpu.CompilerParams(needs_layout_passes=False),
        out_type=jax.ShapeDtypeStruct((NSC, OH * OW), jnp.float32),
        scratch_types=[
            pltpu.VMEM((H * W,), jnp.float32),
            pltpu.VMEM((OH * OW,), jnp.float32),
            pltpu.SemaphoreType.DMA,
        ],
    )(_sc_pool_kernel)


def kernel(inputs):
    x = inputs.reshape(BC, OH, 2, W)
    out_tc = pl.pallas_call(
        _pool_kernel,
        grid=(GRID,),
        in_specs=[pl.BlockSpec((ROWS, OH, 2, W), lambda i: (i, 0, 0, 0))],
        out_specs=pl.BlockSpec((ROWS, OH, OW), lambda i: (i, 0, 0)),
        out_shape=jax.ShapeDtypeStruct((NTC, OH, OW), jnp.float32),
    )(x)
    out_sc = _sc_pool()(inputs.reshape(BC, H * W))
    out = jnp.concatenate([out_tc, out_sc.reshape(NSC, OH, OW)], axis=0)
    return out.reshape(B, C, OH, OW)
